# baseline (speedup 1.0000x reference)
import functools

import jax
import jax.numpy as jnp
from jax import lax
from jax.experimental import pallas as pl
from jax.experimental.pallas import tpu as pltpu
from jax.experimental.pallas import tpu_sc as plsc
from jax.experimental.layout import Format, Layout

_B = 1024
_L = 1024
_E = 64
_BL = _B * _L

_NW = 32
_ROWS_PER_W = _BL // _NW
_C = 128
_NCH = _ROWS_PER_W // _C
_NG = _C // 16

_RSQRT_MAGIC = 0x5F3759DF

_GATHER_DNUMS = lax.GatherDimensionNumbers(
    offset_dims=(), collapsed_slice_dims=(0,), start_index_map=(0,))


def _bcast(v, idx):
    return lax.gather(v, idx[:, None], dimension_numbers=_GATHER_DNUMS,
                      slice_sizes=(1,),
                      mode=lax.GatherScatterMode.PROMISE_IN_BOUNDS)


_XORIDX = None


def _ln_quad(r1d, pe1d, g_regs, b_regs, xor_idx, l_base, q):
    R = 4
    roff = [q * R + rr for rr in range(R)]
    poff = [l_base + q * R + rr for rr in range(R)]
    h = [[r1d[roff[rr], pl.ds(16 * k, 16)] + pe1d[poff[rr], pl.ds(16 * k, 16)]
          for k in range(4)] for rr in range(R)]
    s = [(h[rr][0] + h[rr][1]) + (h[rr][2] + h[rr][3]) for rr in range(R)]
    s2 = [(h[rr][0] * h[rr][0] + h[rr][1] * h[rr][1])
          + (h[rr][2] * h[rr][2] + h[rr][3] * h[rr][3]) for rr in range(R)]
    for m in range(4):
        s = [s[rr] + _bcast(s[rr], xor_idx[m]) for rr in range(R)]
        s2 = [s2[rr] + _bcast(s2[rr], xor_idx[m]) for rr in range(R)]
    mu = [s[rr] * (1.0 / 64.0) for rr in range(R)]
    tv = [s2[rr] * (1.0 / 64.0) - mu[rr] * mu[rr] + 1e-12 for rr in range(R)]
    y = [plsc.bitcast(jnp.int32(_RSQRT_MAGIC) - (plsc.bitcast(t, jnp.int32) >> 1),
                      jnp.float32) for t in tv]
    xh = [t * 0.5 for t in tv]
    for _ in range(2):
        y = [y[rr] * (1.5 - xh[rr] * y[rr] * y[rr]) for rr in range(R)]
    mm = [-(mu[rr] * y[rr]) for rr in range(R)]
    for k in range(4):
        for rr in range(R):
            t = h[rr][k] * y[rr] + mm[rr]
            r1d[roff[rr], pl.ds(16 * k, 16)] = t * g_regs[k] + b_regs[k]


def _ln_chunk(rows, pe1d, g_regs, b_regs, xor_idx, l_base):
    @pl.loop(0, _C // 4)
    def _quad(q):
        _ln_quad(rows, pe1d, g_regs, b_regs, xor_idx, l_base, q)


def _make_sc_call():
    mesh = plsc.VectorSubcoreMesh(core_axis_name="c", subcore_axis_name="s")

    @functools.partial(
        pl.kernel,
        out_type=jax.ShapeDtypeStruct((_B, _L, _E), jnp.float32),
        mesh=mesh,
        compiler_params=pltpu.CompilerParams(
            needs_layout_passes=False, use_tc_tiling_on_sc=False),
        scratch_types=[
            pltpu.VMEM((_L, _E), jnp.float32),
            pltpu.VMEM((_E,), jnp.float32),
            pltpu.VMEM((_E,), jnp.float32),
            pltpu.VMEM((_NCH, _C), jnp.int32),
            pltpu.VMEM((2, _C, _E), jnp.float32),
            pltpu.SemaphoreType.DMA((2,)),
            pltpu.SemaphoreType.DMA((2,)),
        ],
    )
    def sc_embed(x_hbm, table_hbm, pe_hbm, gamma_hbm, beta_hbm, out_hbm,
                 pe_v, g_v, b_v, idx_v, rows_v, gsem, osem):
        wid = lax.axis_index("s") * 2 + lax.axis_index("c")
        base = wid * _ROWS_PER_W

        pltpu.sync_copy(pe_hbm, pe_v)
        pltpu.sync_copy(gamma_hbm, g_v)
        pltpu.sync_copy(beta_hbm, b_v)
        pltpu.sync_copy(x_hbm.at[pl.ds(wid * _NCH, _NCH), :], idx_v)
        g_regs = [g_v[pl.ds(16 * k, 16)] for k in range(4)]
        b_regs = [b_v[pl.ds(16 * k, 16)] for k in range(4)]
        xor_idx = [lax.iota(jnp.int32, 16) ^ m for m in (8, 4, 2, 1)]

        def fire_gather(c, p):
            pltpu.async_copy(table_hbm.at[idx_v.at[c]], rows_v.at[p], gsem.at[p])

        def wait_gather(p):
            pltpu.make_async_copy(
                table_hbm.at[idx_v.at[0]], rows_v.at[p], gsem.at[p]).wait()

        def fire_out(c, p):
            flat = base + c * _C
            pltpu.async_copy(
                rows_v.at[p],
                out_hbm.at[flat // _L, pl.ds(flat % _L, _C), :], osem.at[p])

        def wait_out(p):
            pltpu.make_async_copy(
                rows_v.at[p], out_hbm.at[0, pl.ds(0, _C), :], osem.at[p]).wait()

        fire_gather(0, 0)

        @pl.loop(0, _NCH, step=2)
        def _steady(c0):
            for p in (0, 1):
                c = c0 + p
                q = 1 - p

                @pl.when(c + 1 < _NCH)
                def _prefetch():
                    @pl.when(c >= 1)
                    def _drain_prev_out():
                        wait_out(q)
                    fire_gather(c + 1, q)

                wait_gather(p)
                l_base = (c * _C) & (_L - 1)
                _ln_chunk(rows_v.at[p], pe_v, g_regs, b_regs, xor_idx, l_base)
                fire_out(c, p)

        wait_out(0)
        wait_out(1)

    return sc_embed


_sc_embed = _make_sc_call()


def _kernel_body(x, table, pe, gamma, beta):
    x2 = x.reshape(_BL // _C, _C)
    return _sc_embed(x2, table, pe.reshape(_L, _E), gamma, beta)


_kernel_body.__name__ = "kernel"
_jitted = None


def kernel(x, table, pe, gamma, beta):
    global _jitted
    if _jitted is None:
        try:
            dev = next(iter(x.devices()))
        except Exception:
            dev = jax.devices()[0]
        fmt = Format(Layout(major_to_minor=(0, 1, 2), tiling=None),
                     jax.sharding.SingleDeviceSharding(dev))
        _jitted = jax.jit(_kernel_body, out_shardings=fmt)
    return _jitted(x, table, pe, gamma, beta)


# speedup vs baseline: 1.0299x; 1.0299x over previous
import functools

import jax
import jax.numpy as jnp
from jax import lax
from jax.experimental import pallas as pl
from jax.experimental.pallas import tpu as pltpu
from jax.experimental.pallas import tpu_sc as plsc
from jax.experimental.layout import Format, Layout

_B = 1024
_L = 1024
_E = 64
_BL = _B * _L

_NW = 32
_ROWS_PER_W = _BL // _NW
_C = 128
_NCH = _ROWS_PER_W // _C
_NG = _C // 16

_RSQRT_MAGIC = 0x5F3759DF

_GATHER_DNUMS = lax.GatherDimensionNumbers(
    offset_dims=(), collapsed_slice_dims=(0,), start_index_map=(0,))


def _bcast(v, idx):
    return lax.gather(v, idx[:, None], dimension_numbers=_GATHER_DNUMS,
                      slice_sizes=(1,),
                      mode=lax.GatherScatterMode.PROMISE_IN_BOUNDS)


_XORIDX = None


def _ln_quad(r1d, pe1d, g_regs, b_regs, xor_idx, l_base, q):
    R = 4
    roff = [q * R + rr for rr in range(R)]
    poff = [l_base + q * R + rr for rr in range(R)]
    h = [[r1d[roff[rr], pl.ds(16 * k, 16)] + pe1d[poff[rr], pl.ds(16 * k, 16)]
          for k in range(4)] for rr in range(R)]
    s = [(h[rr][0] + h[rr][1]) + (h[rr][2] + h[rr][3]) for rr in range(R)]
    s2 = [(h[rr][0] * h[rr][0] + h[rr][1] * h[rr][1])
          + (h[rr][2] * h[rr][2] + h[rr][3] * h[rr][3]) for rr in range(R)]
    for m in range(4):
        s = [s[rr] + _bcast(s[rr], xor_idx[m]) for rr in range(R)]
        s2 = [s2[rr] + _bcast(s2[rr], xor_idx[m]) for rr in range(R)]
    mu = [s[rr] * (1.0 / 64.0) for rr in range(R)]
    tv = [s2[rr] * (1.0 / 64.0) - mu[rr] * mu[rr] + 1e-12 for rr in range(R)]
    y = [plsc.bitcast(jnp.int32(_RSQRT_MAGIC) - (plsc.bitcast(t, jnp.int32) >> 1),
                      jnp.float32) for t in tv]
    xh = [t * 0.5 for t in tv]
    for _ in range(2):
        y = [y[rr] * (1.5 - xh[rr] * y[rr] * y[rr]) for rr in range(R)]
    mm = [-(mu[rr] * y[rr]) for rr in range(R)]
    for k in range(4):
        for rr in range(R):
            t = h[rr][k] * y[rr] + mm[rr]
            r1d[roff[rr], pl.ds(16 * k, 16)] = t * g_regs[k] + b_regs[k]


def _ln_chunk(rows, pe1d, g_regs, b_regs, xor_idx, l_base):
    @pl.loop(0, _C // 4, unroll=2)
    def _quad(q):
        _ln_quad(rows, pe1d, g_regs, b_regs, xor_idx, l_base, q)


def _make_sc_call():
    mesh = plsc.VectorSubcoreMesh(core_axis_name="c", subcore_axis_name="s")

    @functools.partial(
        pl.kernel,
        out_type=jax.ShapeDtypeStruct((_B, _L, _E), jnp.float32),
        mesh=mesh,
        compiler_params=pltpu.CompilerParams(
            needs_layout_passes=False, use_tc_tiling_on_sc=False),
        scratch_types=[
            pltpu.VMEM((_L, _E), jnp.float32),
            pltpu.VMEM((_E,), jnp.float32),
            pltpu.VMEM((_E,), jnp.float32),
            pltpu.VMEM((_NCH, _C), jnp.int32),
            pltpu.VMEM((2, _C, _E), jnp.float32),
            pltpu.SemaphoreType.DMA((2,)),
            pltpu.SemaphoreType.DMA((2,)),
        ],
    )
    def sc_embed(x_hbm, table_hbm, pe_hbm, gamma_hbm, beta_hbm, out_hbm,
                 pe_v, g_v, b_v, idx_v, rows_v, gsem, osem):
        wid = lax.axis_index("s") * 2 + lax.axis_index("c")
        base = wid * _ROWS_PER_W

        pltpu.sync_copy(pe_hbm, pe_v)
        pltpu.sync_copy(gamma_hbm, g_v)
        pltpu.sync_copy(beta_hbm, b_v)
        pltpu.sync_copy(x_hbm.at[pl.ds(wid * _NCH, _NCH), :], idx_v)
        g_regs = [g_v[pl.ds(16 * k, 16)] for k in range(4)]
        b_regs = [b_v[pl.ds(16 * k, 16)] for k in range(4)]
        xor_idx = [lax.iota(jnp.int32, 16) ^ m for m in (8, 4, 2, 1)]

        def fire_gather(c, p):
            pltpu.async_copy(table_hbm.at[idx_v.at[c]], rows_v.at[p], gsem.at[p])

        def wait_gather(p):
            pltpu.make_async_copy(
                table_hbm.at[idx_v.at[0]], rows_v.at[p], gsem.at[p]).wait()

        def fire_out(c, p):
            flat = base + c * _C
            pltpu.async_copy(
                rows_v.at[p],
                out_hbm.at[flat // _L, pl.ds(flat % _L, _C), :], osem.at[p])

        def wait_out(p):
            pltpu.make_async_copy(
                rows_v.at[p], out_hbm.at[0, pl.ds(0, _C), :], osem.at[p]).wait()

        fire_gather(0, 0)

        @pl.loop(0, _NCH, step=2)
        def _steady(c0):
            for p in (0, 1):
                c = c0 + p
                q = 1 - p

                @pl.when(c + 1 < _NCH)
                def _prefetch():
                    @pl.when(c >= 1)
                    def _drain_prev_out():
                        wait_out(q)
                    fire_gather(c + 1, q)

                wait_gather(p)
                l_base = (c * _C) & (_L - 1)
                _ln_chunk(rows_v.at[p], pe_v, g_regs, b_regs, xor_idx, l_base)
                fire_out(c, p)

        wait_out(0)
        wait_out(1)

    return sc_embed


_sc_embed = _make_sc_call()


def _kernel_body(x, table, pe, gamma, beta):
    x2 = x.reshape(_BL // _C, _C)
    return _sc_embed(x2, table, pe.reshape(_L, _E), gamma, beta)


_kernel_body.__name__ = "kernel"
_jitted = None


def kernel(x, table, pe, gamma, beta):
    global _jitted
    if _jitted is None:
        try:
            dev = next(iter(x.devices()))
        except Exception:
            dev = jax.devices()[0]
        fmt = Format(Layout(major_to_minor=(0, 1, 2), tiling=None),
                     jax.sharding.SingleDeviceSharding(dev))
        _jitted = jax.jit(_kernel_body, out_shardings=fmt)
    return _jitted(x, table, pe, gamma, beta)


# speedup vs baseline: 1.0441x; 1.0138x over previous
import functools

import jax
import jax.numpy as jnp
from jax import lax
from jax.experimental import pallas as pl
from jax.experimental.pallas import tpu as pltpu
from jax.experimental.pallas import tpu_sc as plsc
from jax.experimental.layout import Format, Layout

_B = 1024
_L = 1024
_E = 64
_BL = _B * _L

_NW = 32
_ROWS_PER_W = _BL // _NW
_C = 128
_NCH = _ROWS_PER_W // _C
_NG = _C // 16

_RSQRT_MAGIC = 0x5F3759DF

_GATHER_DNUMS = lax.GatherDimensionNumbers(
    offset_dims=(), collapsed_slice_dims=(0,), start_index_map=(0,))


def _bcast(v, idx):
    return lax.gather(v, idx[:, None], dimension_numbers=_GATHER_DNUMS,
                      slice_sizes=(1,),
                      mode=lax.GatherScatterMode.PROMISE_IN_BOUNDS)


_XORIDX = None


def _ln_quad(r1d, pe1d, g_regs, b_regs, xor_idx, l_base, q):
    R = 4
    roff = [q * R + rr for rr in range(R)]
    poff = [l_base + q * R + rr for rr in range(R)]
    h = [[r1d[roff[rr], pl.ds(16 * k, 16)] + pe1d[poff[rr], pl.ds(16 * k, 16)]
          for k in range(4)] for rr in range(R)]
    s = [(h[rr][0] + h[rr][1]) + (h[rr][2] + h[rr][3]) for rr in range(R)]
    s2 = [(h[rr][0] * h[rr][0] + h[rr][1] * h[rr][1])
          + (h[rr][2] * h[rr][2] + h[rr][3] * h[rr][3]) for rr in range(R)]
    for m in range(4):
        s = [s[rr] + _bcast(s[rr], xor_idx[m]) for rr in range(R)]
        s2 = [s2[rr] + _bcast(s2[rr], xor_idx[m]) for rr in range(R)]
    mu = [s[rr] * (1.0 / 64.0) for rr in range(R)]
    tv = [s2[rr] * (1.0 / 64.0) - mu[rr] * mu[rr] + 1e-12 for rr in range(R)]
    y = [plsc.bitcast(jnp.int32(_RSQRT_MAGIC) - (plsc.bitcast(t, jnp.int32) >> 1),
                      jnp.float32) for t in tv]
    xh = [t * 0.5 for t in tv]
    for _ in range(2):
        y = [y[rr] * (1.5 - xh[rr] * y[rr] * y[rr]) for rr in range(R)]
    mm = [-(mu[rr] * y[rr]) for rr in range(R)]
    for k in range(4):
        for rr in range(R):
            t = h[rr][k] * y[rr] + mm[rr]
            r1d[roff[rr], pl.ds(16 * k, 16)] = t * g_regs[k] + b_regs[k]


def _ln_chunk(rows, pe1d, g_regs, b_regs, xor_idx, l_base):
    @pl.loop(0, _C // 4, unroll=4)
    def _quad(q):
        _ln_quad(rows, pe1d, g_regs, b_regs, xor_idx, l_base, q)


def _make_sc_call():
    mesh = plsc.VectorSubcoreMesh(core_axis_name="c", subcore_axis_name="s")

    @functools.partial(
        pl.kernel,
        out_type=jax.ShapeDtypeStruct((_B, _L, _E), jnp.float32),
        mesh=mesh,
        compiler_params=pltpu.CompilerParams(
            needs_layout_passes=False, use_tc_tiling_on_sc=False),
        scratch_types=[
            pltpu.VMEM((_L, _E), jnp.float32),
            pltpu.VMEM((_E,), jnp.float32),
            pltpu.VMEM((_E,), jnp.float32),
            pltpu.VMEM((_NCH, _C), jnp.int32),
            pltpu.VMEM((2, _C, _E), jnp.float32),
            pltpu.SemaphoreType.DMA((2,)),
            pltpu.SemaphoreType.DMA((2,)),
        ],
    )
    def sc_embed(x_hbm, table_hbm, pe_hbm, gamma_hbm, beta_hbm, out_hbm,
                 pe_v, g_v, b_v, idx_v, rows_v, gsem, osem):
        wid = lax.axis_index("s") * 2 + lax.axis_index("c")
        base = wid * _ROWS_PER_W

        pltpu.sync_copy(pe_hbm, pe_v)
        pltpu.sync_copy(gamma_hbm, g_v)
        pltpu.sync_copy(beta_hbm, b_v)
        pltpu.sync_copy(x_hbm.at[pl.ds(wid * _NCH, _NCH), :], idx_v)
        g_regs = [g_v[pl.ds(16 * k, 16)] for k in range(4)]
        b_regs = [b_v[pl.ds(16 * k, 16)] for k in range(4)]
        xor_idx = [lax.iota(jnp.int32, 16) ^ m for m in (8, 4, 2, 1)]

        def fire_gather(c, p):
            pltpu.async_copy(table_hbm.at[idx_v.at[c]], rows_v.at[p], gsem.at[p])

        def wait_gather(p):
            pltpu.make_async_copy(
                table_hbm.at[idx_v.at[0]], rows_v.at[p], gsem.at[p]).wait()

        def fire_out(c, p):
            flat = base + c * _C
            pltpu.async_copy(
                rows_v.at[p],
                out_hbm.at[flat // _L, pl.ds(flat % _L, _C), :], osem.at[p])

        def wait_out(p):
            pltpu.make_async_copy(
                rows_v.at[p], out_hbm.at[0, pl.ds(0, _C), :], osem.at[p]).wait()

        fire_gather(0, 0)

        @pl.loop(0, _NCH, step=2)
        def _steady(c0):
            for p in (0, 1):
                c = c0 + p
                q = 1 - p

                @pl.when(c + 1 < _NCH)
                def _prefetch():
                    @pl.when(c >= 1)
                    def _drain_prev_out():
                        wait_out(q)
                    fire_gather(c + 1, q)

                wait_gather(p)
                l_base = (c * _C) & (_L - 1)
                _ln_chunk(rows_v.at[p], pe_v, g_regs, b_regs, xor_idx, l_base)
                fire_out(c, p)

        wait_out(0)
        wait_out(1)

    return sc_embed


_sc_embed = _make_sc_call()


def _kernel_body(x, table, pe, gamma, beta):
    x2 = x.reshape(_BL // _C, _C)
    return _sc_embed(x2, table, pe.reshape(_L, _E), gamma, beta)


_kernel_body.__name__ = "kernel"
_jitted = None


def kernel(x, table, pe, gamma, beta):
    global _jitted
    if _jitted is None:
        try:
            dev = next(iter(x.devices()))
        except Exception:
            dev = jax.devices()[0]
        fmt = Format(Layout(major_to_minor=(0, 1, 2), tiling=None),
                     jax.sharding.SingleDeviceSharding(dev))
        _jitted = jax.jit(_kernel_body, out_shardings=fmt)
    return _jitted(x, table, pe, gamma, beta)
